# Initial kernel scaffold; baseline (speedup 1.0000x reference)
import jax
import jax.numpy as jnp
from jax.experimental import pallas as pl


def kernel(emb, w0, b0, w1, b1, w2, b2):
    raise NotImplementedError("write your pallas kernel here")


# speedup vs baseline: 1.1498x; 1.1498x over previous
import functools

import jax
import jax.numpy as jnp
from jax.experimental import pallas as pl
from jax.experimental.pallas import tpu as pltpu


def _round_up(x, m):
    return ((x + m - 1) // m) * m


def _mlp_kernel(x_ref, w0_ref, b0_ref, w1_ref, b1_ref, w2_ref, b2_ref, o_ref):
    x = x_ref[...].astype(jnp.bfloat16)
    h = jnp.dot(x, w0_ref[...], preferred_element_type=jnp.float32)
    h = jnp.maximum(h + b0_ref[...], 0.0).astype(jnp.bfloat16)
    h = jnp.dot(h, w1_ref[...], preferred_element_type=jnp.float32)
    h = jnp.maximum(h + b1_ref[...], 0.0).astype(jnp.bfloat16)
    o = jnp.dot(h, w2_ref[...], preferred_element_type=jnp.float32)
    o_ref[...] = (o + b2_ref[...]).astype(o_ref.dtype)


def kernel(emb, w0, b0, w1, b1, w2, b2):
    M, K = emb.shape
    N = w2.shape[1]

    tm = 512
    M_pad = _round_up(M, tm)
    x = emb
    if M_pad != M:
        x = jnp.zeros((M_pad, K), emb.dtype).at[:M, :].set(emb)

    w0b = w0.astype(jnp.bfloat16)
    w1b = w1.astype(jnp.bfloat16)
    w2b = w2.astype(jnp.bfloat16)

    def w_spec(shape):
        return pl.BlockSpec(shape, lambda i: (0, 0),
                            pipeline_mode=pl.Buffered(1))

    out = pl.pallas_call(
        _mlp_kernel,
        out_shape=jax.ShapeDtypeStruct((M_pad, N), emb.dtype),
        grid=(M_pad // tm,),
        in_specs=[
            pl.BlockSpec((tm, K), lambda i: (i, 0)),
            w_spec(w0b.shape), w_spec(b0.shape),
            w_spec(w1b.shape), w_spec(b1.shape),
            w_spec(w2b.shape), w_spec(b2.shape),
        ],
        out_specs=pl.BlockSpec((tm, N), lambda i: (i, 0)),
        compiler_params=pltpu.CompilerParams(
            dimension_semantics=("parallel",),
            vmem_limit_bytes=100 * 1024 * 1024,
        ),
    )(x, w0b, b0, w1b, b1, w2b, b2)
    if M_pad != M:
        out = out[:M]
    return out


# speedup vs baseline: 1.2067x; 1.0495x over previous
import functools

import jax
import jax.numpy as jnp
from jax.experimental import pallas as pl
from jax.experimental.pallas import tpu as pltpu


def _round_up(x, m):
    return ((x + m - 1) // m) * m


def _mlp_kernel(x_ref, w0_ref, b0_ref, w1_ref, b1_ref, w2_ref, b2_ref, o_ref):
    x = x_ref[...].astype(jnp.bfloat16)
    h = jnp.dot(x, w0_ref[...], preferred_element_type=jnp.float32)
    h = jnp.maximum(h + b0_ref[...], 0.0).astype(jnp.bfloat16)
    h = jnp.dot(h, w1_ref[...], preferred_element_type=jnp.float32)
    h = jnp.maximum(h + b1_ref[...], 0.0).astype(jnp.bfloat16)
    o = jnp.dot(h, w2_ref[...], preferred_element_type=jnp.float32)
    o_ref[...] = (o + b2_ref[...]).astype(o_ref.dtype)


def kernel(emb, w0, b0, w1, b1, w2, b2):
    M, K = emb.shape
    N = w2.shape[1]

    tm = 1024
    M_pad = _round_up(M, tm)
    x = emb
    if M_pad != M:
        x = jnp.zeros((M_pad, K), emb.dtype).at[:M, :].set(emb)

    w0b = w0.astype(jnp.bfloat16)
    w1b = w1.astype(jnp.bfloat16)
    w2b = w2.astype(jnp.bfloat16)

    def w_spec(shape):
        return pl.BlockSpec(shape, lambda i: (0, 0),
                            pipeline_mode=pl.Buffered(1))

    out = pl.pallas_call(
        _mlp_kernel,
        out_shape=jax.ShapeDtypeStruct((M_pad, N), emb.dtype),
        grid=(M_pad // tm,),
        in_specs=[
            pl.BlockSpec((tm, K), lambda i: (i, 0)),
            w_spec(w0b.shape), w_spec(b0.shape),
            w_spec(w1b.shape), w_spec(b1.shape),
            w_spec(w2b.shape), w_spec(b2.shape),
        ],
        out_specs=pl.BlockSpec((tm, N), lambda i: (i, 0)),
        compiler_params=pltpu.CompilerParams(
            dimension_semantics=("arbitrary",),
            vmem_limit_bytes=100 * 1024 * 1024,
        ),
    )(x, w0b, b0, w1b, b1, w2b, b2)
    if M_pad != M:
        out = out[:M]
    return out
